# Initial kernel scaffold; baseline (speedup 1.0000x reference)
import jax
import jax.numpy as jnp
from jax.experimental import pallas as pl


def kernel(inputs, embed_weight):
    raise NotImplementedError("write your pallas kernel here")


# speedup vs baseline: 1.3921x; 1.3921x over previous
import functools

import jax
import jax.numpy as jnp
from jax import lax
from jax.experimental import pallas as pl
from jax.experimental.pallas import tpu as pltpu
from jax.experimental.pallas import tpu_sc as plsc

B = 16384
S = 52
K = S - 1
D = 128
E = 1000
MAX_NORM = 1.0
EPS = 1e-7

CHUNK = 128


def _renorm_body(w_ref, out_ref):
    w = w_ref[...]
    norms = jnp.sqrt(jnp.sum(w * w, axis=1, keepdims=True))
    scale = jnp.minimum(1.0, MAX_NORM / (norms + EPS))
    out_ref[...] = w * scale


def _renorm_table(w):
    return pl.pallas_call(
        _renorm_body,
        out_shape=jax.ShapeDtypeStruct((E, D), jnp.float32),
    )(w)


def _sc_gather(table, sub_idx, obj_idx, nc, ns):
    nw = nc * ns
    rows = (B * K) // nw
    nchunks = rows // CHUNK
    mesh = plsc.VectorSubcoreMesh(
        core_axis_name="c", subcore_axis_name="s",
        num_cores=nc, num_subcores=ns)

    @functools.partial(
        pl.kernel,
        out_type=(jax.ShapeDtypeStruct((B * K, D), jnp.float32),
                  jax.ShapeDtypeStruct((B * K, D), jnp.float32)),
        mesh=mesh,
        scratch_types=[
            pltpu.VMEM((nchunks, CHUNK), jnp.int32),
            pltpu.VMEM((nchunks, CHUNK), jnp.int32),
            pltpu.VMEM((CHUNK, D), jnp.float32),
            pltpu.VMEM((CHUNK, D), jnp.float32),
            pltpu.SemaphoreType.DMA,
            pltpu.SemaphoreType.DMA,
        ],
    )
    def k(table_hbm, sub_hbm, obj_hbm, sub_out, obj_out,
          sidx_v, oidx_v, sbuf, obuf, ssem, osem):
        wid = lax.axis_index("s") * nc + lax.axis_index("c")
        base = wid * rows
        pltpu.sync_copy(sub_hbm.at[wid], sidx_v)
        pltpu.sync_copy(obj_hbm.at[wid], oidx_v)

        def body(j, carry):
            cs = pltpu.async_copy(table_hbm.at[sidx_v.at[j]], sbuf, ssem)
            co = pltpu.async_copy(table_hbm.at[oidx_v.at[j]], obuf, osem)
            cs.wait()
            pltpu.sync_copy(sbuf, sub_out.at[pl.ds(base + j * CHUNK, CHUNK)])
            co.wait()
            pltpu.sync_copy(obuf, obj_out.at[pl.ds(base + j * CHUNK, CHUNK)])
            return carry

        lax.fori_loop(0, nchunks, body, None)

    return k(table,
             sub_idx.reshape(nw, nchunks, CHUNK),
             obj_idx.reshape(nw, nchunks, CHUNK))


def kernel(inputs, embed_weight):
    scaled = _renorm_table(embed_weight)
    sub_idx = jnp.broadcast_to(inputs[:, :1], (B, K))
    obj_idx = inputs[:, 1:]
    info = plsc.get_sparse_core_info()
    sub, obj = _sc_gather(scaled, sub_idx, obj_idx,
                          info.num_cores, info.num_subcores)
    return sub.reshape(B, K, D), obj.reshape(B, K, D)


# speedup vs baseline: 1.5587x; 1.1197x over previous
import functools

import jax
import jax.numpy as jnp
from jax import lax
from jax.experimental import pallas as pl
from jax.experimental.pallas import tpu as pltpu
from jax.experimental.pallas import tpu_sc as plsc

B = 16384
S = 52
K = S - 1
D = 128
E = 1000
MAX_NORM = 1.0
EPS = 1e-7

CHUNK = 128
GROUP = 2
GR = GROUP * CHUNK


def _renorm_body(w_ref, out_ref):
    w = w_ref[...]
    norms = jnp.sqrt(jnp.sum(w * w, axis=1, keepdims=True))
    scale = jnp.minimum(1.0, MAX_NORM / (norms + EPS))
    out_ref[...] = w * scale


def _renorm_table(w):
    return pl.pallas_call(
        _renorm_body,
        out_shape=jax.ShapeDtypeStruct((E, D), jnp.float32),
    )(w)


def _sc_gather(table, sub_idx, obj_idx, nc, ns):
    nw = nc * ns
    rows = (B * K) // nw
    nchunks = rows // CHUNK
    ngroups = nchunks // GROUP
    mesh = plsc.VectorSubcoreMesh(
        core_axis_name="c", subcore_axis_name="s",
        num_cores=nc, num_subcores=ns)

    @functools.partial(
        pl.kernel,
        out_type=(jax.ShapeDtypeStruct((B * K, D), jnp.float32),
                  jax.ShapeDtypeStruct((B * K, D), jnp.float32)),
        mesh=mesh,
        scratch_types=[
            pltpu.VMEM((nchunks, CHUNK), jnp.int32),
            pltpu.VMEM((2, GR, D), jnp.float32),
            pltpu.SemaphoreType.DMA,
            pltpu.SemaphoreType.DMA,
            pltpu.SemaphoreType.DMA,
            pltpu.SemaphoreType.DMA,
        ],
    )
    def k(table_hbm, sub_hbm, obj_hbm, sub_out, obj_out,
          idxs, bufs, gsem0, gsem1, ssem0, ssem1):
        wid = lax.axis_index("s") * nc + lax.axis_index("c")
        base = wid * rows
        gsem = (gsem0, gsem1)
        ssem = (ssem0, ssem1)

        def phase(idx_hbm, out):
            pltpu.sync_copy(idx_hbm.at[wid], idxs)
            for c in range(GROUP):
                pltpu.async_copy(
                    table_hbm.at[idxs.at[c]],
                    bufs.at[0].at[pl.ds(c * CHUNK, CHUNK)], gsem[0])

            @pl.loop(0, ngroups, step=2)
            def _(tt):
                for p in range(2):
                    t = tt + p
                    q = 1 - p
                    for c in range(GROUP):
                        pltpu.make_async_copy(
                            table_hbm.at[idxs.at[t * GROUP + c]],
                            bufs.at[p].at[pl.ds(c * CHUNK, CHUNK)],
                            gsem[p]).wait()
                    pltpu.async_copy(
                        bufs.at[p], out.at[pl.ds(base + t * GR, GR)], ssem[p])
                    @pl.when(t > 0)
                    def _():
                        pltpu.make_async_copy(
                            bufs.at[q],
                            out.at[pl.ds(base + (t - 1) * GR, GR)],
                            ssem[q]).wait()

                    @pl.when(t + 1 < ngroups)
                    def _():
                        for c in range(GROUP):
                            pltpu.async_copy(
                                table_hbm.at[idxs.at[(t + 1) * GROUP + c]],
                                bufs.at[q].at[pl.ds(c * CHUNK, CHUNK)],
                                gsem[q])

            pltpu.make_async_copy(
                bufs.at[1], out.at[pl.ds(base + (ngroups - 1) * GR, GR)],
                ssem[1]).wait()

        phase(sub_hbm, sub_out)
        phase(obj_hbm, obj_out)

    return k(table,
             sub_idx.reshape(nw, nchunks, CHUNK),
             obj_idx.reshape(nw, nchunks, CHUNK))


def kernel(inputs, embed_weight):
    scaled = _renorm_table(embed_weight)
    sub_idx = jnp.broadcast_to(inputs[:, :1], (B, K))
    obj_idx = inputs[:, 1:]
    info = plsc.get_sparse_core_info()
    sub, obj = _sc_gather(scaled, sub_idx, obj_idx,
                          info.num_cores, info.num_subcores)
    return sub.reshape(B, K, D), obj.reshape(B, K, D)


# speedup vs baseline: 2.6429x; 1.6956x over previous
import functools

import jax
import jax.numpy as jnp
from jax import lax
from jax.experimental import pallas as pl
from jax.experimental.pallas import tpu as pltpu
from jax.experimental.pallas import tpu_sc as plsc

B = 16384
S = 52
K = S - 1
D = 128
E = 1000
MAX_NORM = 1.0
EPS = 1e-7

CHUNK = 128
GROUP = 2
GR = GROUP * CHUNK


def _renorm_body(w_ref, out_ref):
    w = w_ref[...]
    norms = jnp.sqrt(jnp.sum(w * w, axis=1, keepdims=True))
    scale = jnp.minimum(1.0, MAX_NORM / (norms + EPS))
    out_ref[...] = w * scale


def _renorm_table(w):
    return pl.pallas_call(
        _renorm_body,
        out_shape=jax.ShapeDtypeStruct((E, D), jnp.float32),
    )(w)


def _sc_gather(table, sub_idx, obj_idx, nc, ns):
    nw = nc * ns
    rows = (B * K) // nw
    nchunks = rows // CHUNK
    ngroups = nchunks // GROUP
    mesh = plsc.VectorSubcoreMesh(
        core_axis_name="c", subcore_axis_name="s",
        num_cores=nc, num_subcores=ns)

    @functools.partial(
        pl.kernel,
        out_type=(jax.ShapeDtypeStruct((B * K, D), jnp.float32),
                  jax.ShapeDtypeStruct((B * K, D), jnp.float32)),
        mesh=mesh,
        scratch_types=[
            pltpu.VMEM((nchunks, CHUNK), jnp.int32),
            pltpu.VMEM((2, GR, D), jnp.float32),
            pltpu.VMEM_SHARED((E, D), jnp.float32),
            pltpu.SemaphoreType.DMA,
            pltpu.SemaphoreType.DMA,
            pltpu.SemaphoreType.DMA,
            pltpu.SemaphoreType.DMA,
        ],
    )
    def k(table_hbm, sub_hbm, obj_hbm, sub_out, obj_out,
          idxs, bufs, table_sp, gsem0, gsem1, ssem0, ssem1):
        wid = lax.axis_index("s") * nc + lax.axis_index("c")
        base = wid * rows
        gsem = (gsem0, gsem1)
        ssem = (ssem0, ssem1)

        @pl.when(lax.axis_index("s") == 0)
        def _():
            pltpu.sync_copy(table_hbm, table_sp)

        plsc.subcore_barrier()

        def phase(idx_hbm, out):
            pltpu.sync_copy(idx_hbm.at[wid], idxs)
            for c in range(GROUP):
                pltpu.async_copy(
                    table_sp.at[idxs.at[c]],
                    bufs.at[0].at[pl.ds(c * CHUNK, CHUNK)], gsem[0])

            @pl.loop(0, ngroups, step=2)
            def _(tt):
                for p in range(2):
                    t = tt + p
                    q = 1 - p
                    for c in range(GROUP):
                        pltpu.make_async_copy(
                            table_sp.at[idxs.at[t * GROUP + c]],
                            bufs.at[p].at[pl.ds(c * CHUNK, CHUNK)],
                            gsem[p]).wait()
                    pltpu.async_copy(
                        bufs.at[p], out.at[pl.ds(base + t * GR, GR)], ssem[p])
                    @pl.when(t > 0)
                    def _():
                        pltpu.make_async_copy(
                            bufs.at[q],
                            out.at[pl.ds(base + (t - 1) * GR, GR)],
                            ssem[q]).wait()

                    @pl.when(t + 1 < ngroups)
                    def _():
                        for c in range(GROUP):
                            pltpu.async_copy(
                                table_sp.at[idxs.at[(t + 1) * GROUP + c]],
                                bufs.at[q].at[pl.ds(c * CHUNK, CHUNK)],
                                gsem[q])

            pltpu.make_async_copy(
                bufs.at[1], out.at[pl.ds(base + (ngroups - 1) * GR, GR)],
                ssem[1]).wait()

        phase(sub_hbm, sub_out)
        phase(obj_hbm, obj_out)

    return k(table,
             sub_idx.reshape(nw, nchunks, CHUNK),
             obj_idx.reshape(nw, nchunks, CHUNK))


def kernel(inputs, embed_weight):
    scaled = _renorm_table(embed_weight)
    sub_idx = jnp.broadcast_to(inputs[:, :1], (B, K))
    obj_idx = inputs[:, 1:]
    info = plsc.get_sparse_core_info()
    sub, obj = _sc_gather(scaled, sub_idx, obj_idx,
                          info.num_cores, info.num_subcores)
    return sub.reshape(B, K, D), obj.reshape(B, K, D)


# speedup vs baseline: 12.7404x; 4.8207x over previous
import functools

import jax
import jax.numpy as jnp
from jax import lax
from jax.experimental import pallas as pl
from jax.experimental.pallas import tpu as pltpu
from jax.experimental.pallas import tpu_sc as plsc

B = 16384
S = 52
K = S - 1
D = 128
E = 1000
MAX_NORM = 1.0
EPS = 1e-7

CHUNK = 128
GROUP = 2
GR = GROUP * CHUNK


def _renorm_body(w_ref, out_ref):
    w = w_ref[...]
    norms = jnp.sqrt(jnp.sum(w * w, axis=1, keepdims=True))
    scale = jnp.minimum(1.0, MAX_NORM / (norms + EPS))
    out_ref[...] = w * scale


def _renorm_table(w):
    return pl.pallas_call(
        _renorm_body,
        out_shape=jax.ShapeDtypeStruct((E, D), jnp.float32),
    )(w)


def _sc_gather(table, sub_idx, obj_idx, nc, ns):
    nw = nc * ns
    rows = (B * K) // nw
    nchunks = rows // CHUNK
    ngroups = nchunks // GROUP
    mesh = plsc.VectorSubcoreMesh(
        core_axis_name="c", subcore_axis_name="s",
        num_cores=nc, num_subcores=ns)

    @functools.partial(
        pl.kernel,
        out_type=(jax.ShapeDtypeStruct((B * K, D), jnp.float32),
                  jax.ShapeDtypeStruct((B * K, D), jnp.float32)),
        mesh=mesh,
        scratch_types=[
            pltpu.VMEM((nchunks, CHUNK), jnp.int32),
            pltpu.VMEM((2, GR, D), jnp.float32),
            pltpu.VMEM_SHARED((E, D), jnp.float32),
            pltpu.SemaphoreType.DMA,
            pltpu.SemaphoreType.DMA,
            pltpu.SemaphoreType.DMA,
            pltpu.SemaphoreType.DMA,
        ],
    )
    def k(table_hbm, sub_hbm, obj_hbm, sub_out, obj_out,
          idxs, bufs, table_sp, gsem0, gsem1, ssem0, ssem1):
        wid = lax.axis_index("s") * nc + lax.axis_index("c")
        base = wid * rows
        gsem = (gsem0, gsem1)
        ssem = (ssem0, ssem1)

        @pl.when(lax.axis_index("s") == 0)
        def _():
            pltpu.sync_copy(table_hbm, table_sp)

        plsc.subcore_barrier()

        def phase(idx_hbm, out):
            pltpu.sync_copy(idx_hbm.at[wid], idxs)
            for c in range(GROUP):
                pltpu.async_copy(
                    table_sp.at[idxs.at[c]],
                    bufs.at[0].at[pl.ds(c * CHUNK, CHUNK)], gsem[0])

            @pl.loop(0, ngroups, step=2)
            def _(tt):
                for p in range(2):
                    t = tt + p
                    q = 1 - p
                    for c in range(GROUP):
                        pltpu.make_async_copy(
                            table_sp.at[idxs.at[t * GROUP + c]],
                            bufs.at[p].at[pl.ds(c * CHUNK, CHUNK)],
                            gsem[p]).wait()
                    pltpu.async_copy(
                        bufs.at[p], out.at[pl.ds(base + t * GR, GR)], ssem[p])
                    @pl.when(t > 0)
                    def _():
                        pltpu.make_async_copy(
                            bufs.at[q],
                            out.at[pl.ds(base + (t - 1) * GR, GR)],
                            ssem[q]).wait()

                    @pl.when(t + 1 < ngroups)
                    def _():
                        for c in range(GROUP):
                            pltpu.async_copy(
                                table_sp.at[idxs.at[(t + 1) * GROUP + c]],
                                bufs.at[q].at[pl.ds(c * CHUNK, CHUNK)],
                                gsem[q])

            pltpu.make_async_copy(
                bufs.at[1], out.at[pl.ds(base + (ngroups - 1) * GR, GR)],
                ssem[1]).wait()

        phase(sub_hbm, sub_out)
        phase(obj_hbm, obj_out)

    return k(table,
             sub_idx.reshape(nw, nchunks, CHUNK),
             obj_idx.reshape(nw, nchunks, CHUNK))


def kernel(inputs, embed_weight):
    scaled = _renorm_table(embed_weight)
    sub_idx = jnp.broadcast_to(inputs[:, 0][None, :], (K, B))
    obj_idx = inputs[:, 1:].T
    info = plsc.get_sparse_core_info()
    sub, obj = _sc_gather(scaled, sub_idx, obj_idx,
                          info.num_cores, info.num_subcores)
    return (sub.reshape(K, B, D).transpose(1, 0, 2),
            obj.reshape(K, B, D).transpose(1, 0, 2))
